# baseline (speedup 1.0000x reference)
import functools

import jax
import jax.numpy as jnp
from jax import lax
from jax.experimental import pallas as pl
from jax.experimental.pallas import tpu as pltpu
from jax.experimental.pallas import tpu_sc as plsc

_DIM = 64
_LANES = 16
_NCORES = 2
_NSUB = 16
_NW = _NCORES * _NSUB
_GCHUNK = 128


def _make_sc_call(batch):
    bpw = batch // _NW
    nch = bpw // _GCHUNK
    mesh = plsc.VectorSubcoreMesh(core_axis_name="c", subcore_axis_name="s")

    @functools.partial(
        pl.kernel,
        mesh=mesh,
        out_type=jax.ShapeDtypeStruct((_NW, _LANES), jnp.float32),
        compiler_params=pltpu.CompilerParams(use_tc_tiling_on_sc=False),
        scratch_types=[
            pltpu.VMEM((nch, _GCHUNK), jnp.int32),
            pltpu.VMEM((bpw, _DIM), jnp.float32),
            pltpu.VMEM((bpw, _DIM), jnp.float32),
            pltpu.VMEM((_LANES,), jnp.float32),
            pltpu.SemaphoreType.DMA,
            pltpu.SemaphoreType.DMA,
        ],
    )
    def sc_kernel(x_hbm, y_hbm, centers_hbm, out_hbm, idx_v, x_v, c_v, acc_v,
                  sem_x, sem_g):
        wid = lax.axis_index("s") * _NCORES + lax.axis_index("c")
        base = wid * bpw

        pltpu.sync_copy(y_hbm.at[pl.ds(wid * nch, nch)], idx_v)

        cp_x = pltpu.async_copy(x_hbm.at[pl.ds(base, bpw)], x_v, sem_x)
        gathers = []
        for j in range(nch):
            gathers.append(
                pltpu.async_copy(
                    centers_hbm.at[idx_v.at[j]],
                    c_v.at[pl.ds(j * _GCHUNK, _GCHUNK)],
                    sem_g,
                )
            )
        cp_x.wait()
        for g in gathers:
            g.wait()

        zeros = jnp.zeros((_LANES,), jnp.float32)

        def body(r, accs):
            new = []
            for k in range(_DIM // _LANES):
                d = x_v[r, pl.ds(k * _LANES, _LANES)] - c_v[r, pl.ds(k * _LANES, _LANES)]
                new.append(accs[k] + d * d)
            return tuple(new)

        accs = lax.fori_loop(0, bpw, body, (zeros, zeros, zeros, zeros))
        acc_v[...] = accs[0] + accs[1] + accs[2] + accs[3]
        pltpu.sync_copy(acc_v, out_hbm.at[wid])

    return sc_kernel


def kernel(x, y, centers):
    batch, dim = x.shape
    y2 = y.reshape(batch // _GCHUNK, _GCHUNK).astype(jnp.int32)
    partials = _make_sc_call(batch)(x, y2, centers)
    return jnp.sum(partials) / (batch * dim)
